# baseline (speedup 1.0000x reference)
import functools

import jax
import jax.numpy as jnp
from jax import lax
from jax.experimental import pallas as pl
from jax.experimental.pallas import tpu as pltpu
from jax.experimental.pallas import tpu_sc as plsc

_B = 16384
_D = 32
_H = 64
_PK = 4
_W = _PK * _D
_CH = 128

_info = plsc.get_sparse_core_info()
_NC, _NS = _info.num_cores, _info.num_subcores
_NW = _NC * _NS
_BPW = _B // _NW
_NCH = _BPW // _CH

_mesh = plsc.VectorSubcoreMesh(core_axis_name="c", subcore_axis_name="s")


@functools.partial(
    pl.kernel,
    mesh=_mesh,
    out_type=(
        jax.ShapeDtypeStruct((_B, _W), jnp.float32),
        jax.ShapeDtypeStruct((_B, _W), jnp.float32),
    ),
    scratch_types=[
        pltpu.VMEM((_NCH, _CH), jnp.int32),
        pltpu.VMEM((_BPW, _W), jnp.float32),
        pltpu.SemaphoreType.DMA,
    ],
)
def _gather_sc(ujdx_hbm, ijdx_hbm, utab_hbm, itab_hbm, u_out, i_out,
               jdx_v, rows_v, sem):
    wid = lax.axis_index("s") * _NC + lax.axis_index("c")
    base = wid * _BPW

    for tab_hbm, jdx_hbm, out_hbm in (
        (utab_hbm, ujdx_hbm, u_out),
        (itab_hbm, ijdx_hbm, i_out),
    ):
        pltpu.sync_copy(jdx_hbm.at[wid], jdx_v)
        copies = []
        for c in range(_NCH):
            copies.append(pltpu.async_copy(
                tab_hbm.at[jdx_v.at[c]],
                rows_v.at[pl.ds(c * _CH, _CH)], sem))
        for cp in copies:
            cp.wait()
        pltpu.sync_copy(rows_v, out_hbm.at[pl.ds(base, _BPW)])


def _mlp_tc(u_ref, i_ref, uq_ref, iq_ref, w1_ref, b1_ref, w2_ref, b2_ref,
            o_ref):
    uq = uq_ref[...]
    iq = iq_ref[...]
    u = jnp.zeros((u_ref.shape[0], _D), jnp.float32)
    i = jnp.zeros((i_ref.shape[0], _D), jnp.float32)
    for q in range(_PK):
        u = jnp.where(uq == q, u_ref[:, q * _D:(q + 1) * _D], u)
        i = jnp.where(iq == q, i_ref[:, q * _D:(q + 1) * _D], i)
    h = jnp.dot(u, w1_ref[0:_D, :], preferred_element_type=jnp.float32)
    h = h + jnp.dot(i, w1_ref[_D:, :], preferred_element_type=jnp.float32)
    h = jnp.maximum(h + b1_ref[...], 0.0)
    logits = jnp.dot(h, w2_ref[...], preferred_element_type=jnp.float32)
    o_ref[...] = jax.nn.sigmoid(logits + b2_ref[...])


_BLK = 2048


def kernel(user_indices, item_indices, user_table, item_table, W1, b1, W2, b2):
    uidx = user_indices.astype(jnp.int32)
    iidx = item_indices.astype(jnp.int32)
    ujdx = (uidx >> 2).reshape(_NW, _NCH, _CH)
    ijdx = (iidx >> 2).reshape(_NW, _NCH, _CH)
    uq = (uidx & 3).reshape(_B, 1)
    iq = (iidx & 3).reshape(_B, 1)
    utab_p = user_table.reshape(-1, _W)
    itab_p = item_table.reshape(-1, _W)
    u_blk, i_blk = _gather_sc(ujdx, ijdx, utab_p, itab_p)

    out = pl.pallas_call(
        _mlp_tc,
        grid=(_B // _BLK,),
        in_specs=[
            pl.BlockSpec((_BLK, _W), lambda b: (b, 0)),
            pl.BlockSpec((_BLK, _W), lambda b: (b, 0)),
            pl.BlockSpec((_BLK, 1), lambda b: (b, 0)),
            pl.BlockSpec((_BLK, 1), lambda b: (b, 0)),
            pl.BlockSpec((2 * _D, _H), lambda b: (0, 0)),
            pl.BlockSpec((1, _H), lambda b: (0, 0)),
            pl.BlockSpec((_H, 1), lambda b: (0, 0)),
            pl.BlockSpec((1, 1), lambda b: (0, 0)),
        ],
        out_specs=pl.BlockSpec((_BLK, 1), lambda b: (b, 0)),
        out_shape=jax.ShapeDtypeStruct((_B, 1), jnp.float32),
    )(u_blk, i_blk, uq, iq, W1, b1.reshape(1, _H), W2, b2.reshape(1, 1))
    return out


# speedup vs baseline: 1.6563x; 1.6563x over previous
import functools

import jax
import jax.numpy as jnp
from jax import lax
from jax.experimental import pallas as pl
from jax.experimental.pallas import tpu as pltpu
from jax.experimental.pallas import tpu_sc as plsc

_B = 16384
_D = 32
_H = 64

_info = plsc.get_sparse_core_info()
_NC, _NS = _info.num_cores, _info.num_subcores
_NW = _NC * _NS
_BPW = _B // _NW

_mesh = plsc.VectorSubcoreMesh(core_axis_name="c", subcore_axis_name="s")


@functools.partial(
    pl.kernel,
    mesh=_mesh,
    out_type=jax.ShapeDtypeStruct((_B, _D), jnp.float32),
    scratch_types=[
        pltpu.VMEM((_BPW,), jnp.int32),
        pltpu.VMEM((_BPW, _D), jnp.float32),
        pltpu.SemaphoreType.DMA,
    ],
)
def _gather_sc(idx_hbm, tab_hbm, out_hbm, idx_v, rows_v, sem):
    wid = lax.axis_index("s") * _NC + lax.axis_index("c")
    base = wid * _BPW

    pltpu.sync_copy(idx_hbm.at[wid], idx_v)

    def row_body(g, carry):
        vec = idx_v[pl.ds(g * 16, 16)]
        for l in range(16):
            idx = vec[l]
            pltpu.async_copy(tab_hbm.at[pl.ds(idx, 1)],
                             rows_v.at[pl.ds(g * 16 + l, 1)], sem)
        return carry

    lax.fori_loop(0, _BPW // 16, row_body, 0)
    pltpu.make_async_copy(tab_hbm.at[pl.ds(0, _BPW)], rows_v, sem).wait()
    pltpu.sync_copy(rows_v, out_hbm.at[pl.ds(base, _BPW)])


def _mlp_tc(u_ref, i_ref, w1_ref, b1_ref, w2_ref, b2_ref, o_ref):
    h = jnp.dot(u_ref[...], w1_ref[0:_D, :], preferred_element_type=jnp.float32)
    h = h + jnp.dot(i_ref[...], w1_ref[_D:, :],
                    preferred_element_type=jnp.float32)
    h = jnp.maximum(h + b1_ref[...], 0.0)
    logits = jnp.dot(h, w2_ref[...], preferred_element_type=jnp.float32)
    o_ref[...] = jax.nn.sigmoid(logits + b2_ref[...])


_BLK = 2048


def kernel(user_indices, item_indices, user_table, item_table, W1, b1, W2, b2):
    uidx = user_indices.astype(jnp.int32).reshape(_NW, _BPW)
    iidx = item_indices.astype(jnp.int32).reshape(_NW, _BPW)
    i_emb = _gather_sc(iidx, item_table)
    u_emb = _gather_sc(uidx, user_table)

    out = pl.pallas_call(
        _mlp_tc,
        grid=(_B // _BLK,),
        in_specs=[
            pl.BlockSpec((_BLK, _D), lambda b: (b, 0)),
            pl.BlockSpec((_BLK, _D), lambda b: (b, 0)),
            pl.BlockSpec((2 * _D, _H), lambda b: (0, 0)),
            pl.BlockSpec((1, _H), lambda b: (0, 0)),
            pl.BlockSpec((_H, 1), lambda b: (0, 0)),
            pl.BlockSpec((1, 1), lambda b: (0, 0)),
        ],
        out_specs=pl.BlockSpec((_BLK, 1), lambda b: (b, 0)),
        out_shape=jax.ShapeDtypeStruct((_B, 1), jnp.float32),
    )(u_emb, i_emb, W1, b1.reshape(1, _H), W2, b2.reshape(1, 1))
    return out
